# Initial kernel scaffold; baseline (speedup 1.0000x reference)
import jax
import jax.numpy as jnp
from jax.experimental import pallas as pl


def kernel(h, edge_index, W, b, gamma, beta):
    raise NotImplementedError("write your pallas kernel here")


# speedup vs baseline: 3.0058x; 3.0058x over previous
import functools

import jax
import jax.numpy as jnp
from jax import lax
from jax.experimental import pallas as pl
from jax.experimental.pallas import tpu as pltpu
from jax.experimental.pallas import tpu_sc as plsc

N_NODES = 10000
N_EDGES = 320000
IN_DIM = 128
HIDDEN_DIM = 128
EPS = 1e-5

NC = 2
NS = 16
NW = NC * NS
C = 128
K = 80
E_PAD = NW * K * C
ACC_ROWS = N_NODES + 112
ZROWS = ACC_ROWS // NS


def _sc_aggregate(h, src, dst, zeros):
    mesh = plsc.VectorSubcoreMesh(core_axis_name="c", subcore_axis_name="s")

    @functools.partial(
        pl.kernel,
        out_type=jax.ShapeDtypeStruct((NC, ACC_ROWS, IN_DIM), jnp.float32),
        mesh=mesh,
        scratch_types=[
            pltpu.VMEM((K, C), jnp.int32),
            pltpu.VMEM((K, C), jnp.int32),
            pltpu.VMEM((C, IN_DIM), jnp.float32),
            pltpu.VMEM_SHARED((ACC_ROWS, IN_DIM), jnp.float32),
            pltpu.SemaphoreType.DMA,
        ],
    )
    def sc_kernel(h_hbm, src_hbm, dst_hbm, z_hbm, out_hbm,
                  src_v, dst_v, rows_v, acc_sh, sem):
        c = lax.axis_index("c")
        s = lax.axis_index("s")
        w = c * NS + s
        pltpu.sync_copy(z_hbm.at[pl.ds(s * ZROWS, ZROWS)],
                        acc_sh.at[pl.ds(s * ZROWS, ZROWS)])
        pltpu.sync_copy(src_hbm.at[w], src_v)
        pltpu.sync_copy(dst_hbm.at[w], dst_v)
        plsc.subcore_barrier()

        def body(j, carry):
            pltpu.async_copy(h_hbm.at[src_v.at[j]], rows_v, sem).wait()
            pltpu.sync_copy(rows_v, acc_sh.at[dst_v.at[j]], add=True)
            return carry

        lax.fori_loop(0, K, body, 0)
        plsc.subcore_barrier()
        pltpu.sync_copy(acc_sh.at[pl.ds(s * ZROWS, ZROWS)],
                        out_hbm.at[c, pl.ds(s * ZROWS, ZROWS)])

    return sc_kernel(h, src, dst, zeros)


def _tc_finish(partials, W, b, gamma, beta):
    def body(p_ref, w_ref, b_ref, g_ref, be_ref, o_ref):
        agg = p_ref[0, :N_NODES, :] + p_ref[1, :N_NODES, :]
        out = lax.dot_general(agg, w_ref[...], (((1,), (1,)), ((), ())),
                              preferred_element_type=jnp.float32)
        out = out + b_ref[...]
        mean = jnp.mean(out, axis=0, keepdims=True)
        var = jnp.mean((out - mean) ** 2, axis=0, keepdims=True)
        o_ref[...] = (out - mean) * lax.rsqrt(var + EPS) * g_ref[...] + be_ref[...]

    return pl.pallas_call(
        body,
        out_shape=jax.ShapeDtypeStruct((N_NODES, HIDDEN_DIM), jnp.float32),
    )(partials, W, b.reshape(1, HIDDEN_DIM), gamma.reshape(1, HIDDEN_DIM),
      beta.reshape(1, HIDDEN_DIM))


def kernel(h, edge_index, W, b, gamma, beta):
    src = edge_index[0].astype(jnp.int32)
    dst = edge_index[1].astype(jnp.int32)
    pad = E_PAD - N_EDGES
    src = jnp.concatenate([src, jnp.zeros((pad,), jnp.int32)])
    dst = jnp.concatenate([dst, jnp.full((pad,), N_NODES, jnp.int32)])
    src = src.reshape(NW, K, C)
    dst = dst.reshape(NW, K, C)
    zeros = jnp.zeros((ACC_ROWS, IN_DIM), jnp.float32)
    partials = _sc_aggregate(h, src, dst, zeros)
    return _tc_finish(partials, W, b, gamma, beta)


# speedup vs baseline: 3.3696x; 1.1210x over previous
import functools

import jax
import jax.numpy as jnp
from jax import lax
from jax.experimental import pallas as pl
from jax.experimental.pallas import tpu as pltpu
from jax.experimental.pallas import tpu_sc as plsc

N_NODES = 10000
N_EDGES = 320000
IN_DIM = 128
HIDDEN_DIM = 128
EPS = 1e-5

NC = 2
NS = 16
NW = NC * NS
C = 128
K = 80
E_PAD = NW * K * C
ACC_ROWS = N_NODES + 112
ZROWS = ACC_ROWS // NS


def _sc_aggregate(h, eidx, zeros):
    mesh = plsc.VectorSubcoreMesh(core_axis_name="c", subcore_axis_name="s")

    @functools.partial(
        pl.kernel,
        out_type=jax.ShapeDtypeStruct((NC, ACC_ROWS, IN_DIM), jnp.float32),
        mesh=mesh,
        scratch_types=[
            pltpu.VMEM((2, 2, C), jnp.int32),
            pltpu.VMEM((2, C, IN_DIM), jnp.float32),
            pltpu.VMEM_SHARED((ACC_ROWS, IN_DIM), jnp.float32),
            pltpu.SemaphoreType.DMA,
        ],
    )
    def sc_kernel(h_hbm, eidx_hbm, z_hbm, out_hbm, idx_v, rows_v, acc_sh, sem):
        c = lax.axis_index("c")
        s = lax.axis_index("s")
        w = c * NS + s
        pltpu.sync_copy(z_hbm.at[pl.ds(s * ZROWS, ZROWS)],
                        acc_sh.at[pl.ds(s * ZROWS, ZROWS)])
        plsc.subcore_barrier()

        pltpu.sync_copy(eidx_hbm.at[w, 0], idx_v.at[0])
        pltpu.async_copy(h_hbm.at[idx_v.at[0, 0]], rows_v.at[0], sem)

        def body(j, carry):
            cur = lax.rem(j, 2)
            nxt = lax.rem(j + 1, 2)

            @pl.when(j + 1 < K)
            def _():
                pltpu.sync_copy(eidx_hbm.at[w, j + 1], idx_v.at[nxt])
                pltpu.async_copy(h_hbm.at[idx_v.at[nxt, 0]], rows_v.at[nxt],
                                 sem)

            pltpu.make_async_copy(h_hbm.at[idx_v.at[cur, 0]], rows_v.at[cur],
                                  sem).wait()
            pltpu.sync_copy(rows_v.at[cur], acc_sh.at[idx_v.at[cur, 1]],
                            add=True)
            return carry

        lax.fori_loop(0, K, body, 0)
        plsc.subcore_barrier()
        pltpu.sync_copy(acc_sh.at[pl.ds(s * ZROWS, ZROWS)],
                        out_hbm.at[c, pl.ds(s * ZROWS, ZROWS)])

    return sc_kernel(h, eidx, zeros)


def _tc_finish(partials, W, b, gamma, beta):
    def body(p_ref, w_ref, b_ref, g_ref, be_ref, o_ref):
        agg = p_ref[0, :N_NODES, :] + p_ref[1, :N_NODES, :]
        out = lax.dot_general(agg, w_ref[...], (((1,), (1,)), ((), ())),
                              preferred_element_type=jnp.float32)
        out = out + b_ref[...]
        mean = jnp.mean(out, axis=0, keepdims=True)
        var = jnp.mean((out - mean) ** 2, axis=0, keepdims=True)
        o_ref[...] = (out - mean) * lax.rsqrt(var + EPS) * g_ref[...] + be_ref[...]

    return pl.pallas_call(
        body,
        out_shape=jax.ShapeDtypeStruct((N_NODES, HIDDEN_DIM), jnp.float32),
    )(partials, W, b.reshape(1, HIDDEN_DIM), gamma.reshape(1, HIDDEN_DIM),
      beta.reshape(1, HIDDEN_DIM))


def kernel(h, edge_index, W, b, gamma, beta):
    src = edge_index[0].astype(jnp.int32)
    dst = edge_index[1].astype(jnp.int32)
    pad = E_PAD - N_EDGES
    src = jnp.concatenate([src, jnp.zeros((pad,), jnp.int32)])
    dst = jnp.concatenate([dst, jnp.full((pad,), N_NODES, jnp.int32)])
    eidx = jnp.stack([src.reshape(NW, K, C), dst.reshape(NW, K, C)], axis=2)
    zeros = jnp.zeros((ACC_ROWS, IN_DIM), jnp.float32)
    partials = _sc_aggregate(h, eidx, zeros)
    return _tc_finish(partials, W, b, gamma, beta)


# speedup vs baseline: 6.5790x; 1.9525x over previous
import functools

import jax
import jax.numpy as jnp
from jax import lax
from jax.experimental import pallas as pl
from jax.experimental.pallas import tpu as pltpu
from jax.experimental.pallas import tpu_sc as plsc

N_NODES = 10000
N_EDGES = 320000
IN_DIM = 128
HIDDEN_DIM = 128
EPS = 1e-5

NC = 2
NS = 16
NW = NC * NS
C = 112
K = 90
E_PAD = NW * K * C
ACC_ROWS = N_NODES + 112
ZROWS = ACC_ROWS // NS


def _sc_aggregate(h, eidx, zeros):
    mesh = plsc.VectorSubcoreMesh(core_axis_name="c", subcore_axis_name="s")

    @functools.partial(
        pl.kernel,
        out_type=jax.ShapeDtypeStruct((NC, ACC_ROWS, IN_DIM), jnp.float32),
        mesh=mesh,
        scratch_types=[
            pltpu.VMEM((4, 2, C), jnp.int32),
            pltpu.VMEM((3, C, IN_DIM), jnp.float32),
            pltpu.VMEM_SHARED((ACC_ROWS, IN_DIM), jnp.float32),
            pltpu.SemaphoreType.DMA,
            pltpu.SemaphoreType.DMA,
        ],
    )
    def sc_kernel(h_hbm, eidx_hbm, z_hbm, out_hbm, idx_v, rows_v, acc_sh,
                  sem_g, sem_i):
        c = lax.axis_index("c")
        s = lax.axis_index("s")
        w = c * NS + s
        pltpu.sync_copy(z_hbm.at[pl.ds(s * ZROWS, ZROWS)],
                        acc_sh.at[pl.ds(s * ZROWS, ZROWS)])
        plsc.subcore_barrier()

        pltpu.sync_copy(eidx_hbm.at[w, 0], idx_v.at[0])
        pltpu.sync_copy(eidx_hbm.at[w, 1], idx_v.at[1])
        pltpu.async_copy(h_hbm.at[idx_v.at[0, 0]], rows_v.at[0], sem_g)
        pltpu.async_copy(h_hbm.at[idx_v.at[1, 0]], rows_v.at[1], sem_g)
        pltpu.async_copy(eidx_hbm.at[w, 2], idx_v.at[2], sem_i)

        def body(j, carry):
            cur = lax.rem(j, 3)
            cur4 = lax.rem(j, 4)
            nx2 = lax.rem(j + 2, 3)
            nx2_4 = lax.rem(j + 2, 4)
            nx3_4 = lax.rem(j + 3, 4)

            @pl.when(j + 2 < K)
            def _():
                pltpu.make_async_copy(eidx_hbm.at[w, j + 2], idx_v.at[nx2_4],
                                      sem_i).wait()
                pltpu.async_copy(h_hbm.at[idx_v.at[nx2_4, 0]], rows_v.at[nx2],
                                 sem_g)

            @pl.when(j + 3 < K)
            def _():
                pltpu.async_copy(eidx_hbm.at[w, j + 3], idx_v.at[nx3_4], sem_i)

            pltpu.make_async_copy(h_hbm.at[idx_v.at[cur4, 0]], rows_v.at[cur],
                                  sem_g).wait()
            pltpu.sync_copy(rows_v.at[cur], acc_sh.at[idx_v.at[cur4, 1]],
                            add=True)
            return carry

        lax.fori_loop(0, K, body, 0)
        plsc.subcore_barrier()
        pltpu.sync_copy(acc_sh.at[pl.ds(s * ZROWS, ZROWS)],
                        out_hbm.at[c, pl.ds(s * ZROWS, ZROWS)])

    return sc_kernel(h, eidx, zeros)


def _tc_finish(partials, W, b, gamma, beta):
    def body(p_ref, w_ref, b_ref, g_ref, be_ref, o_ref):
        agg = p_ref[0, :N_NODES, :] + p_ref[1, :N_NODES, :]
        out = lax.dot_general(agg, w_ref[...], (((1,), (1,)), ((), ())),
                              preferred_element_type=jnp.float32)
        out = out + b_ref[...]
        mean = jnp.mean(out, axis=0, keepdims=True)
        var = jnp.mean((out - mean) ** 2, axis=0, keepdims=True)
        o_ref[...] = (out - mean) * lax.rsqrt(var + EPS) * g_ref[...] + be_ref[...]

    return pl.pallas_call(
        body,
        out_shape=jax.ShapeDtypeStruct((N_NODES, HIDDEN_DIM), jnp.float32),
    )(partials, W, b.reshape(1, HIDDEN_DIM), gamma.reshape(1, HIDDEN_DIM),
      beta.reshape(1, HIDDEN_DIM))


def kernel(h, edge_index, W, b, gamma, beta):
    src = edge_index[0].astype(jnp.int32)
    dst = edge_index[1].astype(jnp.int32)
    pad = E_PAD - N_EDGES
    src = jnp.concatenate([src, jnp.zeros((pad,), jnp.int32)])
    dst = jnp.concatenate([dst, jnp.full((pad,), N_NODES, jnp.int32)])
    eidx = jnp.stack([src.reshape(NW, K, C), dst.reshape(NW, K, C)], axis=2)
    zeros = jnp.zeros((ACC_ROWS, IN_DIM), jnp.float32)
    partials = _sc_aggregate(h, eidx, zeros)
    return _tc_finish(partials, W, b, gamma, beta)


# speedup vs baseline: 7.9054x; 1.2016x over previous
import functools

import jax
import jax.numpy as jnp
from jax import lax
from jax.experimental import pallas as pl
from jax.experimental.pallas import tpu as pltpu
from jax.experimental.pallas import tpu_sc as plsc

N_NODES = 10000
N_EDGES = 320000
IN_DIM = 128
HIDDEN_DIM = 128
EPS = 1e-5

NC = 2
NS = 16
NW = NC * NS
C = 64
K = 157
E_PAD = NW * K * C
NB = 5
NI = NB + 1
ACC_ROWS = N_NODES + 112
ZROWS = ACC_ROWS // NS


def _sc_aggregate(h, eidx, zeros):
    mesh = plsc.VectorSubcoreMesh(core_axis_name="c", subcore_axis_name="s")

    @functools.partial(
        pl.kernel,
        out_type=jax.ShapeDtypeStruct((NC, ACC_ROWS, IN_DIM), jnp.float32),
        mesh=mesh,
        scratch_types=[
            pltpu.VMEM((NI, 2, C), jnp.int32),
            pltpu.VMEM((NB, C, IN_DIM), jnp.float32),
            pltpu.VMEM_SHARED((ACC_ROWS, IN_DIM), jnp.float32),
            pltpu.SemaphoreType.DMA,
            pltpu.SemaphoreType.DMA,
        ],
    )
    def sc_kernel(h_hbm, eidx_hbm, z_hbm, out_hbm, idx_v, rows_v, acc_sh,
                  sem_g, sem_i):
        c = lax.axis_index("c")
        s = lax.axis_index("s")
        w = c * NS + s
        pltpu.sync_copy(z_hbm.at[pl.ds(s * ZROWS, ZROWS)],
                        acc_sh.at[pl.ds(s * ZROWS, ZROWS)])
        plsc.subcore_barrier()

        for i in range(NB - 1):
            pltpu.sync_copy(eidx_hbm.at[w, i], idx_v.at[i])
            pltpu.async_copy(h_hbm.at[idx_v.at[i, 0]], rows_v.at[i], sem_g)
        pltpu.async_copy(eidx_hbm.at[w, NB - 1], idx_v.at[NB - 1], sem_i)

        def body(j, carry):
            cur = lax.rem(j, NB)
            curi = lax.rem(j, NI)
            nxg = lax.rem(j + NB - 1, NB)
            nxgi = lax.rem(j + NB - 1, NI)
            nxi = lax.rem(j + NB, NI)

            @pl.when(j + NB - 1 < K)
            def _():
                pltpu.make_async_copy(eidx_hbm.at[w, j + NB - 1],
                                      idx_v.at[nxgi], sem_i).wait()
                pltpu.async_copy(h_hbm.at[idx_v.at[nxgi, 0]], rows_v.at[nxg],
                                 sem_g)

            @pl.when(j + NB < K)
            def _():
                pltpu.async_copy(eidx_hbm.at[w, j + NB], idx_v.at[nxi], sem_i)

            pltpu.make_async_copy(h_hbm.at[idx_v.at[curi, 0]], rows_v.at[cur],
                                  sem_g).wait()
            pltpu.sync_copy(rows_v.at[cur], acc_sh.at[idx_v.at[curi, 1]],
                            add=True)
            return carry

        lax.fori_loop(0, K, body, 0)
        plsc.subcore_barrier()
        pltpu.sync_copy(acc_sh.at[pl.ds(s * ZROWS, ZROWS)],
                        out_hbm.at[c, pl.ds(s * ZROWS, ZROWS)])

    return sc_kernel(h, eidx, zeros)


def _tc_finish(partials, W, b, gamma, beta):
    def body(p_ref, w_ref, b_ref, g_ref, be_ref, o_ref):
        agg = p_ref[0, :N_NODES, :] + p_ref[1, :N_NODES, :]
        out = lax.dot_general(agg, w_ref[...], (((1,), (1,)), ((), ())),
                              preferred_element_type=jnp.float32)
        out = out + b_ref[...]
        mean = jnp.mean(out, axis=0, keepdims=True)
        var = jnp.mean((out - mean) ** 2, axis=0, keepdims=True)
        o_ref[...] = (out - mean) * lax.rsqrt(var + EPS) * g_ref[...] + be_ref[...]

    return pl.pallas_call(
        body,
        out_shape=jax.ShapeDtypeStruct((N_NODES, HIDDEN_DIM), jnp.float32),
    )(partials, W, b.reshape(1, HIDDEN_DIM), gamma.reshape(1, HIDDEN_DIM),
      beta.reshape(1, HIDDEN_DIM))


def kernel(h, edge_index, W, b, gamma, beta):
    src = edge_index[0].astype(jnp.int32)
    dst = edge_index[1].astype(jnp.int32)
    pad = E_PAD - N_EDGES
    src = jnp.concatenate([src, jnp.zeros((pad,), jnp.int32)])
    dst = jnp.concatenate([dst, jnp.full((pad,), N_NODES, jnp.int32)])
    eidx = jnp.stack([src.reshape(NW, K, C), dst.reshape(NW, K, C)], axis=2)
    zeros = jnp.zeros((ACC_ROWS, IN_DIM), jnp.float32)
    partials = _sc_aggregate(h, eidx, zeros)
    return _tc_finish(partials, W, b, gamma, beta)
